# Initial kernel scaffold; baseline (speedup 1.0000x reference)
import jax
import jax.numpy as jnp
from jax.experimental import pallas as pl


def kernel(z, embedding_weight):
    raise NotImplementedError("write your pallas kernel here")


# speedup vs baseline: 1.4848x; 1.4848x over previous
import jax
import jax.numpy as jnp
from jax import lax
from jax.experimental import pallas as pl
from jax.experimental.pallas import tpu as pltpu

N = 100000
Q = 1024
D = 128
BLK = 2000
T = N // BLK
EPS = 1e-8
BIG = 2**30


def _body(zt_ref, w_ref, out_ref, znt_ref, max_ref, idx_ref):
    i = pl.program_id(0)

    @pl.when(i == 0)
    def _init():
        zt = zt_ref[...]
        znorm = jnp.maximum(jnp.sqrt(jnp.sum(zt * zt, axis=0, keepdims=True)), EPS)
        znt_ref[...] = zt / znorm
        max_ref[...] = jnp.full((1, Q), -jnp.inf, jnp.float32)
        idx_ref[...] = jnp.zeros((1, Q), jnp.int32)

    w = w_ref[...]
    wnorm = jnp.maximum(jnp.sqrt(jnp.sum(w * w, axis=1, keepdims=True)), EPS)
    wn = w / wnorm
    scores = lax.dot_general(
        wn, znt_ref[...],
        (((1,), (0,)), ((), ())),
        preferred_element_type=jnp.float32,
        precision=lax.Precision.DEFAULT,
    )
    m = jnp.max(scores, axis=0, keepdims=True)
    rows = lax.broadcasted_iota(jnp.int32, (BLK, Q), 0) + i * BLK
    cand = jnp.min(jnp.where(scores == m, rows, BIG), axis=0, keepdims=True)
    better = m > max_ref[...]
    idx_ref[...] = jnp.where(better, cand, idx_ref[...])
    max_ref[...] = jnp.where(better, m, max_ref[...])

    @pl.when(i == T - 1)
    def _fin():
        out_ref[...] = idx_ref[...]


def kernel(z, embedding_weight):
    zt = z.T
    out = pl.pallas_call(
        _body,
        grid=(T,),
        in_specs=[
            pl.BlockSpec((D, Q), lambda i: (0, 0)),
            pl.BlockSpec((BLK, D), lambda i: (i, 0)),
        ],
        out_specs=pl.BlockSpec((1, Q), lambda i: (0, 0)),
        out_shape=jax.ShapeDtypeStruct((1, Q), jnp.int32),
        scratch_shapes=[
            pltpu.VMEM((D, Q), jnp.float32),
            pltpu.VMEM((1, Q), jnp.float32),
            pltpu.VMEM((1, Q), jnp.int32),
        ],
    )(zt, embedding_weight)
    return out.reshape(Q)


# speedup vs baseline: 2.3573x; 1.5876x over previous
import jax
import jax.numpy as jnp
from jax import lax
from jax.experimental import pallas as pl
from jax.experimental.pallas import tpu as pltpu

N = 100000
Q = 1024
D = 128
BLK = 2000
T = N // BLK
EPS = 1e-8
BIG = 2**30


def _body(zt_ref, w_ref, out_ref, znt_ref, max_ref, idx_ref):
    i = pl.program_id(0)

    @pl.when(i == 0)
    def _init():
        zt = zt_ref[...]
        znorm = jnp.maximum(jnp.sqrt(jnp.sum(zt * zt, axis=0, keepdims=True)), EPS)
        znt_ref[...] = zt / znorm
        max_ref[...] = jnp.full((1, Q), -jnp.inf, jnp.float32)
        idx_ref[...] = jnp.zeros((1, Q), jnp.int32)

    w = w_ref[...]
    wnorm = jnp.maximum(jnp.sqrt(jnp.sum(w * w, axis=1, keepdims=True)), EPS)
    wn = w / wnorm
    scores = lax.dot_general(
        wn, znt_ref[...],
        (((1,), (0,)), ((), ())),
        preferred_element_type=jnp.float32,
        precision=lax.Precision.DEFAULT,
    )

    scores3 = scores.reshape(BLK // 8, 8, Q)
    run = scores3[0]
    ridx = jnp.zeros((8, Q), jnp.int32)
    for r in range(1, BLK // 8):
        sv = scores3[r]
        gt = sv > run
        run = jnp.maximum(run, sv)
        ridx = jnp.where(gt, r, ridx)
    rid = ridx * 8 + lax.broadcasted_iota(jnp.int32, (8, Q), 0)
    m = jnp.max(run, axis=0, keepdims=True)
    cand = jnp.min(jnp.where(run == m, rid, BIG), axis=0, keepdims=True) + i * BLK
    better = m > max_ref[...]
    idx_ref[...] = jnp.where(better, cand, idx_ref[...])
    max_ref[...] = jnp.where(better, m, max_ref[...])

    @pl.when(i == T - 1)
    def _fin():
        out_ref[...] = idx_ref[...]


def kernel(z, embedding_weight):
    zt = z.T
    out = pl.pallas_call(
        _body,
        grid=(T,),
        in_specs=[
            pl.BlockSpec((D, Q), lambda i: (0, 0)),
            pl.BlockSpec((BLK, D), lambda i: (i, 0)),
        ],
        out_specs=pl.BlockSpec((1, Q), lambda i: (0, 0)),
        out_shape=jax.ShapeDtypeStruct((1, Q), jnp.int32),
        scratch_shapes=[
            pltpu.VMEM((D, Q), jnp.float32),
            pltpu.VMEM((1, Q), jnp.float32),
            pltpu.VMEM((1, Q), jnp.int32),
        ],
    )(zt, embedding_weight)
    return out.reshape(Q)
